# baseline (speedup 1.0000x reference)
import functools
import jax
import jax.numpy as jnp
from jax import lax
from jax.experimental import pallas as pl
from jax.experimental.pallas import tpu as pltpu, tpu_sc as plsc

BS = 512


def _add_body(pos_ref, x_ref, out_ref):
    out_ref[...] = x_ref[...] + pos_ref[...][None]


def _make_sc_gather(S, d3, n_rows):
    info = plsc.get_sparse_core_info()
    nw = info.num_cores * info.num_subcores
    rows_per_w = 3 * (S // nw)
    mesh = plsc.VectorSubcoreMesh(core_axis_name="c", subcore_axis_name="s")

    @functools.partial(
        pl.kernel, mesh=mesh,
        out_type=jax.ShapeDtypeStruct((3 * S, d3), jnp.float32),
        scratch_types=[
            pltpu.VMEM((rows_per_w,), jnp.int32),
            pltpu.VMEM((rows_per_w, d3), jnp.float32),
            pltpu.SemaphoreType.DMA,
        ],
    )
    def sc_gather(table_hbm, idx_hbm, pos_hbm, idx_v, rows_v, sem):
        wid = lax.axis_index("s") * info.num_cores + lax.axis_index("c")
        base = wid * rows_per_w
        pltpu.sync_copy(idx_hbm.at[pl.ds(base, rows_per_w)], idx_v)
        n_chunks = rows_per_w // 128
        copies = []
        for j in range(n_chunks):
            copies.append(pltpu.async_copy(
                table_hbm.at[idx_v.at[pl.ds(128 * j, 128)]],
                rows_v.at[pl.ds(128 * j, 128)], sem))
        for c in copies:
            c.wait()
        pltpu.sync_copy(rows_v, pos_hbm.at[pl.ds(base, rows_per_w)])

    return sc_gather


def kernel(x, src_tgt, emb_x, emb_y, emb_z, src_pos_x, src_pos_y, src_pos_z):
    B, S, D = x.shape
    d3 = emb_x.shape[1]
    nx, ny, nz = emb_x.shape[0], emb_y.shape[0], emb_z.shape[0]

    is_src = (src_tgt != 0)
    sx = jnp.concatenate([jnp.array([nx - 1], jnp.int32), src_pos_x])[:S]
    sy = jnp.concatenate([jnp.array([ny - 1], jnp.int32), src_pos_y])[:S]
    sz = jnp.concatenate([jnp.array([nz - 1], jnp.int32), src_pos_z])[:S]
    px = jnp.where(is_src, src_pos_x, sx)
    py = jnp.where(is_src, src_pos_y, sy) + nx
    pz = jnp.where(is_src, src_pos_z, sz) + nx + ny

    table = jnp.concatenate([emb_x, emb_y, emb_z], axis=0)
    idx = jnp.stack([px, py, pz], axis=1).reshape(-1)

    pos = _make_sc_gather(S, d3, nx + ny + nz)(table, idx)
    pos = pos.reshape(S, D)

    nb = S // BS
    out = pl.pallas_call(
        _add_body,
        grid=(nb,),
        in_specs=[
            pl.BlockSpec((BS, D), lambda i: (i, 0)),
            pl.BlockSpec((B, BS, D), lambda i: (0, i, 0)),
        ],
        out_specs=pl.BlockSpec((B, BS, D), lambda i: (0, i, 0)),
        out_shape=jax.ShapeDtypeStruct((B, S, D), jnp.float32),
    )(pos, x)
    return out
